# baseline (speedup 1.0000x reference)
import jax
import jax.numpy as jnp
from jax import lax
from jax.experimental import pallas as pl
from jax.experimental.pallas import tpu as pltpu
from jax.experimental.pallas import tpu_sc as plsc

_BM = 16384


def _sc_gather(pos_table_flat, positions):
    info = plsc.get_sparse_core_info()
    nc, ns, nl = info.num_cores, info.num_subcores, info.num_lanes
    nw = nc * ns
    n = positions.shape[0]
    v = pos_table_flat.shape[0]
    per_w = n // nw
    mesh = plsc.VectorSubcoreMesh(core_axis_name="c", subcore_axis_name="s")

    def body(table_hbm, idx_hbm, out_hbm, tab_v, idx_v, rows_v):
        wid = lax.axis_index("s") * nc + lax.axis_index("c")
        base = wid * per_w
        pltpu.sync_copy(table_hbm, tab_v)
        pltpu.sync_copy(idx_hbm.at[pl.ds(base, per_w)], idx_v)
        for j in range(per_w // nl):
            iv = idx_v[pl.ds(j * nl, nl)]
            rows_v[pl.ds(j * nl, nl)] = plsc.load_gather(tab_v, [iv])
        pltpu.sync_copy(rows_v, out_hbm.at[pl.ds(base, per_w)])

    run = pl.kernel(
        body,
        mesh=mesh,
        compiler_params=pltpu.CompilerParams(needs_layout_passes=False),
        out_type=jax.ShapeDtypeStruct((n,), jnp.float32),
        scratch_types=[
            pltpu.VMEM((v,), jnp.float32),
            pltpu.VMEM((per_w,), jnp.int32),
            pltpu.VMEM((per_w,), jnp.float32),
        ],
    )
    return run(pos_table_flat, positions)


def _add_body(x_ref, pos_ref, o_ref):
    reps, _ = pos_ref.shape
    p = jnp.tile(pos_ref[...], (_BM // reps, 1))
    o_ref[...] = x_ref[...] + p


def kernel(inputs, pos_table, positions):
    B, S, _ = inputs.shape
    R = B * S // 128
    reps = S // 128
    pos_emb = _sc_gather(pos_table.reshape(S), positions)
    x2 = inputs.reshape(R, 128)
    pos_tile = pos_emb.reshape(reps, 128)
    out = pl.pallas_call(
        _add_body,
        grid=(R // _BM,),
        in_specs=[
            pl.BlockSpec((_BM, 128), lambda i: (i, 0)),
            pl.BlockSpec((reps, 128), lambda i: (0, 0)),
        ],
        out_specs=pl.BlockSpec((_BM, 128), lambda i: (i, 0)),
        out_shape=jax.ShapeDtypeStruct((R, 128), jnp.float32),
    )(x2, pos_tile)
    return out.reshape(B, S, 1)


# speedup vs baseline: 1.2509x; 1.2509x over previous
import jax
import jax.numpy as jnp
from jax.experimental import pallas as pl
from jax.experimental.pallas import tpu as pltpu

_BM = 16384


def _add_body(x_ref, tab_ref, idx_ref, o_ref, pos_ref):
    reps, L = pos_ref.shape

    @pl.when(pl.program_id(0) == 0)
    def _gather():
        idx = idx_ref[...]
        tab = tab_ref[...]
        r = jax.lax.shift_right_logical(idx, 7)
        c = jnp.bitwise_and(idx, L - 1)
        acc = jnp.zeros((reps, L), jnp.float32)
        for r0 in range(reps):
            row = jax.lax.broadcast_in_dim(tab[r0, :], (reps, L), (1,))
            g = jnp.take_along_axis(row, c, axis=1)
            acc = jnp.where(r == r0, g, acc)
        pos_ref[...] = acc

    p = jnp.tile(pos_ref[...], (_BM // reps, 1))
    o_ref[...] = x_ref[...] + p


def kernel(inputs, pos_table, positions):
    B, S, _ = inputs.shape
    R = B * S // 128
    reps = S // 128
    x2 = inputs.reshape(R, 128)
    tab_tile = pos_table.reshape(reps, 128)
    idx_tile = positions.reshape(reps, 128)
    out = pl.pallas_call(
        _add_body,
        grid=(R // _BM,),
        in_specs=[
            pl.BlockSpec((_BM, 128), lambda i: (i, 0)),
            pl.BlockSpec((reps, 128), lambda i: (0, 0)),
            pl.BlockSpec((reps, 128), lambda i: (0, 0)),
        ],
        out_specs=pl.BlockSpec((_BM, 128), lambda i: (i, 0)),
        out_shape=jax.ShapeDtypeStruct((R, 128), jnp.float32),
        scratch_shapes=[pltpu.VMEM((reps, 128), jnp.float32)],
    )(x2, tab_tile, idx_tile)
    return out.reshape(B, S, 1)


# speedup vs baseline: 1.2524x; 1.0012x over previous
import jax
import jax.numpy as jnp
from jax.experimental import pallas as pl
from jax.experimental.pallas import tpu as pltpu

_BM = 16384


def _add_body(x_ref, tab_ref, idx_ref, o_ref, pos_ref):
    reps, L = pos_ref.shape

    @pl.when(pl.program_id(0) == 0)
    def _gather():
        idx = idx_ref[...]
        tab = tab_ref[...]
        r = jax.lax.shift_right_logical(idx, (L - 1).bit_length())
        c = jnp.bitwise_and(idx, L - 1)
        acc = jnp.zeros((reps, L), jnp.float32)
        for r0 in range(reps):
            row = jax.lax.broadcast_in_dim(tab[r0, :], (reps, L), (1,))
            g = jnp.take_along_axis(row, c, axis=1)
            acc = jnp.where(r == r0, g, acc)
        pos_ref[...] = acc

    p = jnp.tile(pos_ref[...], (_BM // reps, 1))
    o_ref[...] = x_ref[...] + p


def kernel(inputs, pos_table, positions):
    B, S, _ = inputs.shape
    R = B * S // 128
    reps = S // 128
    x2 = inputs.reshape(R, 128)
    tab_tile = pos_table.reshape(reps, 128)
    idx_tile = positions.astype(jnp.int32).reshape(reps, 128)
    out = pl.pallas_call(
        _add_body,
        grid=(R // _BM,),
        in_specs=[
            pl.BlockSpec((_BM, 128), lambda i: (i, 0)),
            pl.BlockSpec((reps, 128), lambda i: (0, 0)),
            pl.BlockSpec((reps, 128), lambda i: (0, 0)),
        ],
        out_specs=pl.BlockSpec((_BM, 128), lambda i: (i, 0)),
        out_shape=jax.ShapeDtypeStruct((R, 128), jnp.float32),
        scratch_shapes=[pltpu.VMEM((reps, 128), jnp.float32)],
    )(x2, tab_tile, idx_tile)
    return out.reshape(B, S, 1)
